# Initial kernel scaffold; baseline (speedup 1.0000x reference)
import functools

import jax
import jax.numpy as jnp
from jax import lax
from jax.experimental import pallas as pl
from jax.experimental.pallas import tpu as pltpu
from jax.experimental.pallas import tpu_sc as plsc

N = 10000
E = 320000
D = 128
NC = 2
NS = 16
NW = NC * NS
EPW = E // NW
K = 200
NCHUNK = EPW // K
NPAD = 10240
RPT = NPAD // NS

_mesh = plsc.VectorSubcoreMesh(core_axis_name="c", subcore_axis_name="s")


@functools.partial(
    pl.kernel,
    out_type=jax.ShapeDtypeStruct((NC, NPAD), jnp.float32),
    mesh=_mesh,
    scratch_types=[
        pltpu.VMEM((K,), jnp.int32),
        pltpu.VMEM((K,), jnp.float32),
        pltpu.VMEM((RPT,), jnp.float32),
        pltpu.VMEM_SHARED((NPAD,), jnp.float32),
    ],
)
def _deg_kernel(dst_hbm, out_hbm, dbuf, ones, zbuf, deg_sh):
    c = lax.axis_index("c")
    s = lax.axis_index("s")
    wid = s * NC + c
    base = wid * EPW

    def _init(i, _):
        ones[pl.ds(i * 16, 16)] = jnp.ones((16,), jnp.float32)
        return 0

    lax.fori_loop(0, K // 16, _init, 0)

    def _zero(i, _):
        zbuf[pl.ds(i * 16, 16)] = jnp.zeros((16,), jnp.float32)
        return 0

    lax.fori_loop(0, RPT // 16, _zero, 0)
    pltpu.sync_copy(zbuf, deg_sh.at[pl.ds(s * RPT, RPT)])
    plsc.subcore_barrier()

    def _step(i, _):
        pltpu.sync_copy(dst_hbm.at[pl.ds(base + i * K, K)], dbuf)
        pltpu.sync_copy(ones, deg_sh.at[dbuf], add=True)
        return 0

    lax.fori_loop(0, NCHUNK, _step, 0)
    plsc.subcore_barrier()
    pltpu.sync_copy(deg_sh.at[pl.ds(s * RPT, RPT)],
                    out_hbm.at[c, pl.ds(s * RPT, RPT)])


@functools.partial(
    pl.kernel,
    out_type=jax.ShapeDtypeStruct((NC, NPAD, D), jnp.float32),
    mesh=_mesh,
    scratch_types=[
        pltpu.VMEM((K,), jnp.int32),
        pltpu.VMEM((K,), jnp.int32),
        pltpu.VMEM((K, D), jnp.float32),
        pltpu.VMEM((64, D), jnp.float32),
        pltpu.VMEM_SHARED((NPAD, D), jnp.float32),
        pltpu.SemaphoreType.DMA,
    ],
)
def _scatter_kernel(xs_hbm, src_hbm, dst_hbm, out_hbm,
                    sbuf, dbuf, rows, zrows, acc_sh, gsem):
    c = lax.axis_index("c")
    s = lax.axis_index("s")
    wid = s * NC + c
    base = wid * EPW

    def _zero(j, _):
        r = j // 8
        col = (j % 8) * 16
        zrows[r, pl.ds(col, 16)] = jnp.zeros((16,), jnp.float32)
        return 0

    lax.fori_loop(0, 64 * 8, _zero, 0)
    for t in range(RPT // 64):
        pltpu.sync_copy(zrows, acc_sh.at[pl.ds(s * RPT + t * 64, 64), :])
    plsc.subcore_barrier()

    def _step(i, _):
        pltpu.sync_copy(src_hbm.at[pl.ds(base + i * K, K)], sbuf)
        pltpu.sync_copy(dst_hbm.at[pl.ds(base + i * K, K)], dbuf)
        pltpu.async_copy(xs_hbm.at[sbuf], rows, gsem).wait()
        pltpu.sync_copy(rows, acc_sh.at[dbuf], add=True)
        return 0

    lax.fori_loop(0, NCHUNK, _step, 0)
    plsc.subcore_barrier()
    pltpu.sync_copy(acc_sh.at[pl.ds(s * RPT, RPT), :],
                    out_hbm.at[c, pl.ds(s * RPT, RPT), :])


BR = 400
GRID = N // BR


def _k1_body(x_ref, w_ref, degr_ref, out_ref):
    deg = degr_ref[:, 0] + degr_ref[:, 1] + 1.0
    dinv = lax.rsqrt(deg)
    xw = jnp.dot(x_ref[...], w_ref[...], preferred_element_type=jnp.float32)
    out_ref[...] = xw * dinv[:, None]


def _k3_body(acc_ref, xs_ref, degr_ref, w_ref, b_ref, out_ref):
    deg = degr_ref[:, 0] + degr_ref[:, 1] + 1.0
    dinv = lax.rsqrt(deg)[:, None]
    h = jnp.maximum(
        dinv * (acc_ref[0] + acc_ref[1] + xs_ref[...]) + b_ref[...], 0.0)
    out_ref[...] = jnp.dot(
        h, w_ref[...], preferred_element_type=jnp.float32) * dinv


def _k5_body(acc_ref, xs_ref, degr_ref, b2_ref, wf1_ref, bf1_ref,
             wf2_ref, bf2_ref, wt_ref, bt_ref, fx_ref, y_ref):
    deg = degr_ref[:, 0] + degr_ref[:, 1] + 1.0
    dinv = lax.rsqrt(deg)[:, None]
    g = jnp.maximum(
        dinv * (acc_ref[0] + acc_ref[1] + xs_ref[...]) + b2_ref[...], 0.0)
    fx = jnp.maximum(
        jnp.dot(g, wf1_ref[...], preferred_element_type=jnp.float32)
        + bf1_ref[...], 0.0)
    fx_ref[...] = fx
    h2 = jnp.maximum(
        jnp.dot(fx, wf2_ref[...], preferred_element_type=jnp.float32)
        + bf2_ref[...], 0.0)
    y_ref[...] = jnp.dot(
        h2, wt_ref[...], preferred_element_type=jnp.float32) + bt_ref[...]


_row_spec = pl.BlockSpec((BR, D), lambda i: (i, 0))
_deg_spec = pl.BlockSpec((BR, 2), lambda i: (i, 0))
_acc_spec = pl.BlockSpec((NC, BR, D), lambda i: (0, i, 0))
_w_spec = pl.BlockSpec((D, D), lambda i: (0, 0))
_b_spec = pl.BlockSpec((1, D), lambda i: (0, 0))

_k1 = pl.pallas_call(
    _k1_body,
    grid=(GRID,),
    in_specs=[_row_spec, _w_spec, _deg_spec],
    out_specs=_row_spec,
    out_shape=jax.ShapeDtypeStruct((N, D), jnp.float32),
)

_k3 = pl.pallas_call(
    _k3_body,
    grid=(GRID,),
    in_specs=[_acc_spec, _row_spec, _deg_spec, _w_spec, _b_spec],
    out_specs=_row_spec,
    out_shape=jax.ShapeDtypeStruct((N, D), jnp.float32),
)

_k5 = pl.pallas_call(
    _k5_body,
    grid=(GRID,),
    in_specs=[_acc_spec, _row_spec, _deg_spec, _b_spec,
              _w_spec, _b_spec, _w_spec, _b_spec, _w_spec, _b_spec],
    out_specs=[_row_spec, _row_spec],
    out_shape=[jax.ShapeDtypeStruct((N, D), jnp.float32),
               jax.ShapeDtypeStruct((N, D), jnp.float32)],
)


def kernel(x, edge_index, W1, b1, W2, b2, Wf1, bf1, Wf2, bf2, Wt0, bt0,
           Wt1, bt1):
    src = edge_index[0]
    dst = edge_index[1]

    degp = _deg_kernel(dst)
    degr = jnp.transpose(degp)

    xs1 = _k1(x, W1, degr)
    acc1 = _scatter_kernel(xs1, src, dst)
    xs2 = _k3(acc1, xs1, degr, W2, b1.reshape(1, D))
    acc2 = _scatter_kernel(xs2, src, dst)

    n_t0 = Wt0.shape[1]
    n_t1 = Wt1.shape[1]
    wt = jnp.zeros((D, D), jnp.float32)
    wt = wt.at[:, :n_t0].set(Wt0).at[:, n_t0:n_t0 + n_t1].set(Wt1)
    bt = jnp.zeros((1, D), jnp.float32)
    bt = bt.at[0, :n_t0].set(bt0).at[0, n_t0:n_t0 + n_t1].set(bt1)

    fx, ypad = _k5(acc2, xs2, degr, b2.reshape(1, D),
                   Wf1, bf1.reshape(1, D), Wf2, bf2.reshape(1, D), wt, bt)
    y0 = ypad[:, :n_t0]
    y1 = ypad[:, n_t0:n_t0 + n_t1]
    return (y0, y1, fx)


# speedup vs baseline: 18.4746x; 18.4746x over previous
import functools

import jax
import jax.numpy as jnp
from jax import lax
from jax.experimental import pallas as pl
from jax.experimental.pallas import tpu as pltpu
from jax.experimental.pallas import tpu_sc as plsc

N = 10000
E = 320000
D = 128
NC = 2
NS = 16
NW = NC * NS
EPW = E // NW
K = 200
NCHUNK = EPW // K
NPAD = 10240
RPT = NPAD // NS

_mesh = plsc.VectorSubcoreMesh(core_axis_name="c", subcore_axis_name="s")


@functools.partial(
    pl.kernel,
    out_type=jax.ShapeDtypeStruct((NC, NPAD), jnp.float32),
    mesh=_mesh,
    scratch_types=[
        pltpu.VMEM((K,), jnp.int32),
        pltpu.VMEM((K,), jnp.float32),
        pltpu.VMEM((RPT,), jnp.float32),
        pltpu.VMEM_SHARED((NPAD,), jnp.float32),
    ],
)
def _deg_kernel(dst_hbm, out_hbm, dbuf, ones, zbuf, deg_sh):
    c = lax.axis_index("c")
    s = lax.axis_index("s")
    wid = s * NC + c
    base = wid * EPW

    def _init(i, _):
        ones[pl.ds(i * 16, 16)] = jnp.ones((16,), jnp.float32)
        return 0

    lax.fori_loop(0, K // 16, _init, 0)

    def _zero(i, _):
        zbuf[pl.ds(i * 16, 16)] = jnp.zeros((16,), jnp.float32)
        return 0

    lax.fori_loop(0, RPT // 16, _zero, 0)
    pltpu.sync_copy(zbuf, deg_sh.at[pl.ds(s * RPT, RPT)])
    plsc.subcore_barrier()

    def _step(i, _):
        pltpu.sync_copy(dst_hbm.at[pl.ds(base + i * K, K)], dbuf)
        pltpu.sync_copy(ones, deg_sh.at[dbuf], add=True)
        return 0

    lax.fori_loop(0, NCHUNK, _step, 0)
    plsc.subcore_barrier()
    pltpu.sync_copy(deg_sh.at[pl.ds(s * RPT, RPT)],
                    out_hbm.at[c, pl.ds(s * RPT, RPT)])


@functools.partial(
    pl.kernel,
    out_type=jax.ShapeDtypeStruct((NC, NPAD, D), jnp.float32),
    mesh=_mesh,
    scratch_types=[
        pltpu.VMEM((K,), jnp.int32),
        pltpu.VMEM((K,), jnp.int32),
        pltpu.VMEM((K, D), jnp.float32),
        pltpu.VMEM((64, D), jnp.float32),
        pltpu.VMEM_SHARED((NPAD, D), jnp.float32),
        pltpu.SemaphoreType.DMA,
    ],
)
def _scatter_kernel(xs_hbm, src_hbm, dst_hbm, out_hbm,
                    sbuf, dbuf, rows, zrows, acc_sh, gsem):
    c = lax.axis_index("c")
    s = lax.axis_index("s")
    wid = s * NC + c
    base = wid * EPW

    def _zero(j, _):
        r = j // 8
        col = (j % 8) * 16
        zrows[r, pl.ds(col, 16)] = jnp.zeros((16,), jnp.float32)
        return 0

    lax.fori_loop(0, 64 * 8, _zero, 0)
    for t in range(RPT // 64):
        pltpu.sync_copy(zrows, acc_sh.at[pl.ds(s * RPT + t * 64, 64), :])
    plsc.subcore_barrier()

    def _step(i, _):
        pltpu.sync_copy(src_hbm.at[pl.ds(base + i * K, K)], sbuf)
        pltpu.sync_copy(dst_hbm.at[pl.ds(base + i * K, K)], dbuf)
        pltpu.async_copy(xs_hbm.at[sbuf], rows, gsem).wait()
        pltpu.sync_copy(rows, acc_sh.at[dbuf], add=True)
        return 0

    lax.fori_loop(0, NCHUNK, _step, 0)
    plsc.subcore_barrier()
    pltpu.sync_copy(acc_sh.at[pl.ds(s * RPT, RPT), :],
                    out_hbm.at[c, pl.ds(s * RPT, RPT), :])


BR = 400
GRID = N // BR


def _k1_body(x_ref, w_ref, degr_ref, out_ref):
    deg = degr_ref[:, 0] + degr_ref[:, 1] + 1.0
    dinv = lax.rsqrt(deg)
    xw = jnp.dot(x_ref[...], w_ref[...], preferred_element_type=jnp.float32,
        precision=lax.Precision.HIGHEST)
    out_ref[...] = xw * dinv[:, None]


def _k3_body(acc_ref, xs_ref, degr_ref, w_ref, b_ref, out_ref):
    deg = degr_ref[:, 0] + degr_ref[:, 1] + 1.0
    dinv = lax.rsqrt(deg)[:, None]
    h = jnp.maximum(
        dinv * (acc_ref[0] + acc_ref[1] + xs_ref[...]) + b_ref[...], 0.0)
    out_ref[...] = jnp.dot(
        h, w_ref[...], preferred_element_type=jnp.float32,
        precision=lax.Precision.HIGHEST) * dinv


def _k5_body(acc_ref, xs_ref, degr_ref, b2_ref, wf1_ref, bf1_ref,
             wf2_ref, bf2_ref, wt_ref, bt_ref, fx_ref, y_ref):
    deg = degr_ref[:, 0] + degr_ref[:, 1] + 1.0
    dinv = lax.rsqrt(deg)[:, None]
    g = jnp.maximum(
        dinv * (acc_ref[0] + acc_ref[1] + xs_ref[...]) + b2_ref[...], 0.0)
    fx = jnp.maximum(
        jnp.dot(g, wf1_ref[...], preferred_element_type=jnp.float32,
        precision=lax.Precision.HIGHEST)
        + bf1_ref[...], 0.0)
    fx_ref[...] = fx
    h2 = jnp.maximum(
        jnp.dot(fx, wf2_ref[...], preferred_element_type=jnp.float32,
        precision=lax.Precision.HIGHEST)
        + bf2_ref[...], 0.0)
    y_ref[...] = jnp.dot(
        h2, wt_ref[...], preferred_element_type=jnp.float32,
        precision=lax.Precision.HIGHEST) + bt_ref[...]


_row_spec = pl.BlockSpec((BR, D), lambda i: (i, 0))
_deg_spec = pl.BlockSpec((BR, 2), lambda i: (i, 0))
_acc_spec = pl.BlockSpec((NC, BR, D), lambda i: (0, i, 0))
_w_spec = pl.BlockSpec((D, D), lambda i: (0, 0))
_b_spec = pl.BlockSpec((1, D), lambda i: (0, 0))

_k1 = pl.pallas_call(
    _k1_body,
    grid=(GRID,),
    in_specs=[_row_spec, _w_spec, _deg_spec],
    out_specs=_row_spec,
    out_shape=jax.ShapeDtypeStruct((N, D), jnp.float32),
)

_k3 = pl.pallas_call(
    _k3_body,
    grid=(GRID,),
    in_specs=[_acc_spec, _row_spec, _deg_spec, _w_spec, _b_spec],
    out_specs=_row_spec,
    out_shape=jax.ShapeDtypeStruct((N, D), jnp.float32),
)

_k5 = pl.pallas_call(
    _k5_body,
    grid=(GRID,),
    in_specs=[_acc_spec, _row_spec, _deg_spec, _b_spec,
              _w_spec, _b_spec, _w_spec, _b_spec, _w_spec, _b_spec],
    out_specs=[_row_spec, _row_spec],
    out_shape=[jax.ShapeDtypeStruct((N, D), jnp.float32),
               jax.ShapeDtypeStruct((N, D), jnp.float32)],
)


def kernel(x, edge_index, W1, b1, W2, b2, Wf1, bf1, Wf2, bf2, Wt0, bt0,
           Wt1, bt1):
    src = edge_index[0]
    dst = edge_index[1]

    degp = _deg_kernel(dst)
    degr = jnp.transpose(degp)

    xs1 = _k1(x, W1, degr)
    acc1 = _scatter_kernel(xs1, src, dst)
    xs2 = _k3(acc1, xs1, degr, W2, b1.reshape(1, D))
    acc2 = _scatter_kernel(xs2, src, dst)

    n_t0 = Wt0.shape[1]
    n_t1 = Wt1.shape[1]
    wt = jnp.zeros((D, D), jnp.float32)
    wt = wt.at[:, :n_t0].set(Wt0).at[:, n_t0:n_t0 + n_t1].set(Wt1)
    bt = jnp.zeros((1, D), jnp.float32)
    bt = bt.at[0, :n_t0].set(bt0).at[0, n_t0:n_t0 + n_t1].set(bt1)

    fx, ypad = _k5(acc2, xs2, degr, b2.reshape(1, D),
                   Wf1, bf1.reshape(1, D), Wf2, bf2.reshape(1, D), wt, bt)
    y0 = ypad[:, :n_t0]
    y1 = ypad[:, n_t0:n_t0 + n_t1]
    return (y0, y1, fx)
